# baseline (speedup 1.0000x reference)
import functools

import jax
import jax.numpy as jnp
from jax.experimental import pallas as pl
from jax.experimental.pallas import tpu as pltpu

M, B, DIN, DOUT = 8, 1024, 1024, 1024


def _mm_kernel(x_hbm, w_hbm, b_hbm, o_hbm, xb, wb, bb, ob, fsem, ssem):
    m = pl.program_id(0)
    sl = jax.lax.rem(m, 2)
    nsl = 1 - sl

    def fetch(i, slot):
        pltpu.make_async_copy(x_hbm.at[i], xb.at[slot], fsem.at[slot, 0]).start()
        pltpu.make_async_copy(w_hbm.at[i], wb.at[slot], fsem.at[slot, 1]).start()

    @pl.when(m == 0)
    def _():
        pltpu.make_async_copy(b_hbm, bb, fsem.at[0, 2]).start()
        fetch(0, 0)
        fetch(1, 1)
        pltpu.make_async_copy(b_hbm, bb, fsem.at[0, 2]).wait()

    @pl.when(jnp.logical_and(m >= 1, m < M - 1))
    def _():
        fetch(m + 1, nsl)

    pltpu.make_async_copy(x_hbm.at[m], xb.at[sl], fsem.at[sl, 0]).wait()
    pltpu.make_async_copy(w_hbm.at[m], wb.at[sl], fsem.at[sl, 1]).wait()

    @pl.when(m >= 2)
    def _():
        pltpu.make_async_copy(ob.at[sl], o_hbm.at[m - 2], ssem.at[sl]).wait()

    x = xb[sl].astype(jnp.bfloat16)
    w = wb[sl].astype(jnp.bfloat16)
    acc = jax.lax.dot_general(
        x, w, (((1,), (0,)), ((), ())),
        preferred_element_type=jnp.float32,
    )
    ob[sl] = acc + bb[m]
    pltpu.make_async_copy(ob.at[sl], o_hbm.at[m], ssem.at[sl]).start()

    @pl.when(m == M - 1)
    def _():
        pltpu.make_async_copy(ob.at[nsl], o_hbm.at[m - 1], ssem.at[nsl]).wait()
        pltpu.make_async_copy(ob.at[sl], o_hbm.at[m], ssem.at[sl]).wait()


@functools.partial(jax.jit)
def kernel(inputs, W, b):
    hbm = pl.BlockSpec(memory_space=pltpu.MemorySpace.HBM)
    out = pl.pallas_call(
        _mm_kernel,
        grid=(M,),
        in_specs=[hbm, hbm, hbm],
        out_specs=hbm,
        out_shape=jax.ShapeDtypeStruct((M, B, DOUT), jnp.float32),
        scratch_shapes=[
            pltpu.VMEM((2, B, DIN), jnp.float32),
            pltpu.VMEM((2, DIN, DOUT), jnp.float32),
            pltpu.VMEM((M, 1, DOUT), jnp.float32),
            pltpu.VMEM((2, B, DOUT), jnp.float32),
            pltpu.SemaphoreType.DMA((2, 3)),
            pltpu.SemaphoreType.DMA((2,)),
        ],
        compiler_params=pltpu.CompilerParams(
            dimension_semantics=("arbitrary",),
        ),
    )(inputs, W, b.reshape(M, 1, DOUT))
    return out


# speedup vs baseline: 1.0648x; 1.0648x over previous
import functools

import jax
import jax.numpy as jnp
from jax.experimental import pallas as pl
from jax.experimental.pallas import tpu as pltpu

M, B, DIN, DOUT = 8, 1024, 1024, 1024
MG = 2
NSTEP = M // MG


def _mm_kernel(x_ref, w_ref, b_ref, o_hbm, acc_ref, sems):
    m = pl.program_id(0)
    par = jax.lax.rem(m, 2)

    for j in range(MG):
        @pl.when(m >= 2)
        def _():
            pltpu.make_async_copy(
                acc_ref.at[par, j], o_hbm.at[MG * (m - 2) + j], sems.at[par, j]
            ).wait()

        x = x_ref[j].astype(jnp.bfloat16)
        w = w_ref[j].astype(jnp.bfloat16)
        acc = jax.lax.dot_general(
            x, w, (((1,), (0,)), ((), ())),
            preferred_element_type=jnp.float32,
        )
        acc_ref[par, j] = acc + b_ref[j]
        pltpu.make_async_copy(
            acc_ref.at[par, j], o_hbm.at[MG * m + j], sems.at[par, j]
        ).start()

    @pl.when(m == NSTEP - 1)
    def _():
        for j in range(MG):
            pltpu.make_async_copy(
                acc_ref.at[1 - par, j], o_hbm.at[MG * (m - 1) + j],
                sems.at[1 - par, j],
            ).wait()
            pltpu.make_async_copy(
                acc_ref.at[par, j], o_hbm.at[MG * m + j], sems.at[par, j]
            ).wait()


@functools.partial(jax.jit)
def kernel(inputs, W, b):
    out = pl.pallas_call(
        _mm_kernel,
        grid=(NSTEP,),
        in_specs=[
            pl.BlockSpec((MG, B, DIN), lambda m: (m, 0, 0)),
            pl.BlockSpec((MG, DIN, DOUT), lambda m: (m, 0, 0)),
            pl.BlockSpec((MG, 1, DOUT), lambda m: (m, 0, 0)),
        ],
        out_specs=pl.BlockSpec(memory_space=pltpu.MemorySpace.HBM),
        out_shape=jax.ShapeDtypeStruct((M, B, DOUT), jnp.float32),
        scratch_shapes=[
            pltpu.VMEM((2, MG, B, DOUT), jnp.float32),
            pltpu.SemaphoreType.DMA((2, MG)),
        ],
        compiler_params=pltpu.CompilerParams(
            dimension_semantics=("arbitrary",),
        ),
    )(inputs, W, b.reshape(M, 1, DOUT))
    return out
